# baseline (speedup 1.0000x reference)
import functools

import jax
import jax.numpy as jnp
from jax import lax
from jax.experimental import pallas as pl
from jax.experimental.pallas import tpu as pltpu
from jax.experimental.pallas import tpu_sc as plsc

INPUT_SIZE = 200
EMBED = 64
LANES = 16
NUM_WORKERS = 32
NBUF = 4
CHUNK_A = 128
CHUNK_B = INPUT_SIZE - CHUNK_A


def _pos_encoding(n=10000):
    pos = jnp.arange(INPUT_SIZE, dtype=jnp.float32)[:, None]
    i = jnp.arange(EMBED // 2, dtype=jnp.float32)
    den = jnp.power(jnp.float32(n), 2.0 * i / EMBED)
    P = jnp.zeros((INPUT_SIZE, EMBED), dtype=jnp.float32)
    P = P.at[:, 0::2].set(jnp.sin(pos / den))
    P = P.at[:, 1::2].set(jnp.cos(pos / den))
    return P


@functools.lru_cache(maxsize=None)
def _build(batch, vocab):
    seqs_w = batch // NUM_WORKERS
    n_groups = seqs_w // NBUF
    mesh = plsc.VectorSubcoreMesh(core_axis_name="c", subcore_axis_name="s")

    @functools.partial(
        pl.kernel,
        mesh=mesh,
        compiler_params=pltpu.CompilerParams(use_tc_tiling_on_sc=False),
        out_type=jax.ShapeDtypeStruct((batch, INPUT_SIZE, EMBED), jnp.float32),
        scratch_types=[
            pltpu.VMEM((seqs_w, INPUT_SIZE), jnp.int32),
            pltpu.VMEM((INPUT_SIZE, EMBED), jnp.float32),
            pltpu.VMEM((NBUF, INPUT_SIZE, EMBED), jnp.float32),
        ] + [pltpu.SemaphoreType.DMA] * (2 * NBUF),
    )
    def gather_add(table_hbm, idx_hbm, p_hbm, out_hbm, idx_v, p_v, rows_v,
                   *sems):
        gsems, osems = sems[:NBUF], sems[NBUF:]
        wid = lax.axis_index("s") * 2 + lax.axis_index("c")
        seq0 = wid * seqs_w
        pltpu.sync_copy(idx_hbm.at[pl.ds(seq0, seqs_w)], idx_v)
        pltpu.sync_copy(p_hbm, p_v)

        def fire_gather(s, b):
            pltpu.async_copy(
                table_hbm.at[idx_v.at[s, pl.ds(0, CHUNK_A)]],
                rows_v.at[b, pl.ds(0, CHUNK_A)], gsems[b])
            pltpu.async_copy(
                table_hbm.at[idx_v.at[s, pl.ds(CHUNK_A, CHUNK_B)]],
                rows_v.at[b, pl.ds(CHUNK_A, CHUNK_B)], gsems[b])

        def wait_gather(b):
            pltpu.make_async_copy(
                table_hbm.at[pl.ds(0, INPUT_SIZE)], rows_v.at[b],
                gsems[b]).wait()

        def fire_out(s, b):
            pltpu.async_copy(rows_v.at[b], out_hbm.at[seq0 + s], osems[b])

        def wait_out(b):
            pltpu.make_async_copy(
                rows_v.at[b], out_hbm.at[0], osems[b]).wait()

        for b in range(NBUF):
            fire_gather(b, b)

        def group(g, carry):
            for b in range(NBUF):
                s = g * NBUF + b
                wait_gather(b)

                @plsc.parallel_loop(0, INPUT_SIZE, unroll=4)
                def _(r):
                    for j in range(EMBED // LANES):
                        sl = pl.ds(j * LANES, LANES)
                        plsc.addupdate(rows_v.at[b, r, sl], p_v[r, sl])

                fire_out(s, b)

            @pl.when(g + 1 < n_groups)
            def _():
                for b in range(NBUF):
                    wait_out(b)
                    fire_gather((g + 1) * NBUF + b, b)

            return carry

        lax.fori_loop(0, n_groups, group, 0)
        for b in range(NBUF):
            wait_out(b)

    return gather_add


def kernel(x, table):
    b, l = x.shape
    if x.dtype != jnp.int32:
        x = x.astype(jnp.int32)
    p = _pos_encoding()
    return _build(b, table.shape[0])(table, x, p)


# speedup vs baseline: 1.1682x; 1.1682x over previous
import functools

import jax
import jax.numpy as jnp
from jax import lax
from jax.experimental import pallas as pl
from jax.experimental.pallas import tpu as pltpu
from jax.experimental.pallas import tpu_sc as plsc

INPUT_SIZE = 200
EMBED = 64
PADDED = 128
LANES = 16
NUM_WORKERS = 32
NBUF = 2
CHUNK_A = 128
CHUNK_B = INPUT_SIZE - CHUNK_A


def _pos_encoding(n=10000):
    pos = jnp.arange(INPUT_SIZE, dtype=jnp.float32)[:, None]
    i = jnp.arange(EMBED // 2, dtype=jnp.float32)
    den = jnp.power(jnp.float32(n), 2.0 * i / EMBED)
    P = jnp.zeros((INPUT_SIZE, EMBED), dtype=jnp.float32)
    P = P.at[:, 0::2].set(jnp.sin(pos / den))
    P = P.at[:, 1::2].set(jnp.cos(pos / den))
    return P


@functools.lru_cache(maxsize=None)
def _build(n_rows, vocab):
    rows_w = n_rows // NUM_WORKERS
    seqs_w = rows_w // INPUT_SIZE
    n_groups = seqs_w // NBUF
    batch = n_rows // INPUT_SIZE
    mesh = plsc.VectorSubcoreMesh(core_axis_name="c", subcore_axis_name="s")

    @functools.partial(
        pl.kernel,
        mesh=mesh,
        compiler_params=pltpu.CompilerParams(use_tc_tiling_on_sc=True),
        out_type=jax.ShapeDtypeStruct((batch, INPUT_SIZE, PADDED),
                                      jnp.float32),
        scratch_types=[
            pltpu.VMEM((rows_w,), jnp.int32),
            pltpu.VMEM((INPUT_SIZE, PADDED), jnp.float32),
            pltpu.VMEM((NBUF, INPUT_SIZE, PADDED), jnp.float32),
        ] + [pltpu.SemaphoreType.DMA] * (2 * NBUF),
    )
    def gather_add(table_hbm, idx_hbm, p_hbm, out_hbm, idx_v, p_v, rows_v,
                   *sems):
        gsems, osems = sems[:NBUF], sems[NBUF:]
        wid = lax.axis_index("s") * 2 + lax.axis_index("c")
        base = pl.multiple_of(wid * rows_w, 8)
        seq0 = wid * seqs_w
        pltpu.sync_copy(idx_hbm.at[pl.ds(base, rows_w)], idx_v)
        pltpu.sync_copy(p_hbm, p_v)

        def fire_gather(s, b):
            row0 = pl.multiple_of(s * INPUT_SIZE, 8)
            pltpu.async_copy(
                table_hbm.at[idx_v.at[pl.ds(row0, CHUNK_A)]],
                rows_v.at[b, pl.ds(0, CHUNK_A)], gsems[b])
            pltpu.async_copy(
                table_hbm.at[idx_v.at[pl.ds(row0 + CHUNK_A, CHUNK_B)]],
                rows_v.at[b, pl.ds(CHUNK_A, CHUNK_B)], gsems[b])

        def wait_gather(b):
            pltpu.make_async_copy(
                table_hbm.at[pl.ds(0, INPUT_SIZE)], rows_v.at[b],
                gsems[b]).wait()

        def fire_out(s, b):
            pltpu.async_copy(rows_v.at[b], out_hbm.at[seq0 + s], osems[b])

        def wait_out(b):
            pltpu.make_async_copy(
                rows_v.at[b], out_hbm.at[0], osems[b]).wait()

        for b in range(NBUF):
            fire_gather(b, b)

        def group(g, carry):
            for b in range(NBUF):
                s = g * NBUF + b
                wait_gather(b)

                @plsc.parallel_loop(0, INPUT_SIZE, unroll=4)
                def _(r):
                    for j in range(EMBED // LANES):
                        sl = pl.ds(j * LANES, LANES)
                        plsc.addupdate(rows_v.at[b, r, sl], p_v[r, sl])

                fire_out(s, b)

            @pl.when(g + 1 < n_groups)
            def _():
                for b in range(NBUF):
                    wait_out(b)
                    fire_gather((g + 1) * NBUF + b, b)

            return carry

        lax.fori_loop(0, n_groups, group, 0)
        for b in range(NBUF):
            wait_out(b)

    return gather_add


def kernel(x, table):
    b, l = x.shape
    idx = x.reshape(-1)
    if idx.dtype != jnp.int32:
        idx = idx.astype(jnp.int32)
    table128 = jnp.pad(table, ((0, 0), (0, PADDED - EMBED)))
    p = jnp.pad(_pos_encoding(), ((0, 0), (0, PADDED - EMBED)))
    out = _build(b * l, table.shape[0])(table128, idx, p)
    return out[:, :, :EMBED]
